# baseline (speedup 1.0000x reference)
import math

import jax
import jax.numpy as jnp
from jax.experimental import pallas as pl
from jax.experimental.pallas import tpu as pltpu

_RATES = [1, 2, 4, 8]
_INCH = 64
_OUTCH = 128
_K1 = 16
_STEP = 4
_KMAX = 128


def _graph_positions(r):
    sn = (_K1 // _STEP) * (r - 1 + _STEP)
    n_iter = math.ceil(sn // (r - 1 + _STEP))
    pos = []
    for i in range(n_iter):
        lo = (i + 1) * (r - 1) + i * _STEP
        hi = sn if i == n_iter - 1 else (i + 1) * (r - 1 + _STEP)
        pos.extend(range(lo, hi))
    return pos


def _ann_positions(r):
    if r == 1:
        return list(range(16))
    return [0] + list(range((r - 1) * 16, r * 16 - 1))


def _head(y, idx, positions, g, be):
    sel = idx[:, :, jnp.array(positions, dtype=jnp.int32)]
    y_cn = jnp.transpose(y, (0, 2, 1))
    gf = jax.vmap(lambda yb, ib: yb[:, ib])(y_cn, sel)
    h = gf - y_cn[:, :, :, None]
    mean = jnp.mean(h, axis=(0, 2, 3))
    var = jnp.var(h, axis=(0, 2, 3))
    m = jnp.max(h, axis=3)
    out = jax.nn.relu((m - mean[None, :, None]) / jnp.sqrt(var + 1e-5)[None, :, None]
                      * g[None, :, None] + be[None, :, None])
    return jnp.transpose(out, (0, 2, 1))


_ROWS = 256


def _knn_kernel(xq_ref, xat_ref, out_ref, d2_ref):
    R = xq_ref.shape[1]
    N = xat_ref.shape[2]
    dx = xq_ref[0, :, 0:1] - xat_ref[0, 0:1, :]
    dy = xq_ref[0, :, 1:2] - xat_ref[0, 1:2, :]
    dz = xq_ref[0, :, 2:3] - xat_ref[0, 2:3, :]
    d2_ref[...] = dx * dx + dy * dy + dz * dz
    iota = jax.lax.broadcasted_iota(jnp.int32, (R, N), 1)
    kiota = jax.lax.broadcasted_iota(jnp.int32, (R, _KMAX), 1)

    def body(t, acc):
        d2 = d2_ref[...]
        v = jnp.min(d2, axis=1, keepdims=True)
        ix = jnp.min(jnp.where(d2 == v, iota, N), axis=1, keepdims=True)
        d2_ref[...] = jnp.where(iota == ix, jnp.inf, d2)
        return jnp.where(kiota == t, ix, acc)

    acc = jnp.zeros((R, _KMAX), jnp.int32)
    out_ref[0] = jax.lax.fori_loop(0, _KMAX, body, acc)


def _knn_top128(xyz):
    B, N, _ = xyz.shape
    xyzT = jnp.transpose(xyz, (0, 2, 1))
    return pl.pallas_call(
        _knn_kernel,
        grid=(B, N // _ROWS),
        in_specs=[
            pl.BlockSpec((1, _ROWS, 3), lambda b, i: (b, i, 0)),
            pl.BlockSpec((1, 3, N), lambda b, i: (b, 0, 0)),
        ],
        out_specs=pl.BlockSpec((1, _ROWS, _KMAX), lambda b, i: (b, i, 0)),
        out_shape=jax.ShapeDtypeStruct((B, N, _KMAX), jnp.int32),
        scratch_shapes=[pltpu.VMEM((_ROWS, N), jnp.float32)],
    )(xyz, xyzT)


def _fuse_mm_kernel(f_ref, w_ref, o_ref):
    o_ref[...] = jnp.dot(f_ref[...], w_ref[...],
                         preferred_element_type=jnp.float32)


def kernel(xyz, features, params):
    B, N, _ = xyz.shape
    idx = _knn_top128(xyz)

    oc = _OUTCH // 4
    feat_g = features
    graph_list = []
    for i, r in enumerate(_RATES):
        y = jnp.einsum('bnc,oc->bno', feat_g, params['dg_W%d' % i])
        fg = _head(y, idx, _graph_positions(r),
                   params['dg_g%d' % i], params['dg_be%d' % i])
        feat_g = jnp.concatenate([feat_g, fg], axis=-1)
        graph_list.append(fg)

    feat_a = features
    ann_list = []
    for i, r in enumerate(_RATES):
        y = jnp.einsum('bnc,oc->bno', feat_a, params['ad_W%d' % i])
        fa = _head(y, idx, _ann_positions(r),
                   params['ad_g%d' % i], params['ad_be%d' % i])
        feat_a = jnp.concatenate([feat_a, fa], axis=-1)
        ann_list.append(fa)

    fusion = jnp.concatenate(graph_list + ann_list, axis=-1)

    BLK = 512
    z = pl.pallas_call(
        _fuse_mm_kernel,
        grid=(B * N // BLK,),
        in_specs=[
            pl.BlockSpec((BLK, 2 * _OUTCH), lambda i: (i, 0)),
            pl.BlockSpec((2 * _OUTCH, _OUTCH), lambda i: (0, 0)),
        ],
        out_specs=pl.BlockSpec((BLK, _OUTCH), lambda i: (i, 0)),
        out_shape=jax.ShapeDtypeStruct((B * N, _OUTCH), jnp.float32),
    )(fusion.reshape(B * N, 2 * _OUTCH), params['fuse_W'].T)
    z = z.reshape(B, N, _OUTCH)

    mean = jnp.mean(z, axis=(0, 1))
    var = jnp.var(z, axis=(0, 1))
    h = (z - mean) / jnp.sqrt(var + 1e-5) * params['fuse_g'] + params['fuse_be']
    return jax.nn.relu(h)


# speedup vs baseline: 4.5636x; 4.5636x over previous
import functools
import math

import jax
import jax.numpy as jnp
from jax.experimental import pallas as pl
from jax.experimental.pallas import tpu as pltpu
from jax.experimental.pallas import tpu_sc as plsc

_RATES = [1, 2, 4, 8]
_OUTCH = 128
_K1 = 16
_STEP = 4
_KMAX = 128
_K = 16
_NW = 32
_CHUNK = 128


def _graph_positions(r):
    sn = (_K1 // _STEP) * (r - 1 + _STEP)
    n_iter = math.ceil(sn // (r - 1 + _STEP))
    pos = []
    for i in range(n_iter):
        lo = (i + 1) * (r - 1) + i * _STEP
        hi = sn if i == n_iter - 1 else (i + 1) * (r - 1 + _STEP)
        pos.extend(range(lo, hi))
    return pos


def _ann_positions(r):
    if r == 1:
        return list(range(16))
    return [0] + list(range((r - 1) * 16, r * 16 - 1))


_ROWS = 256


def _knn_kernel(xq_ref, xat_ref, out_ref, d2_ref):
    R = xq_ref.shape[1]
    N = xat_ref.shape[2]
    dx = xq_ref[0, :, 0:1] - xat_ref[0, 0:1, :]
    dy = xq_ref[0, :, 1:2] - xat_ref[0, 1:2, :]
    dz = xq_ref[0, :, 2:3] - xat_ref[0, 2:3, :]
    d2_ref[...] = dx * dx + dy * dy + dz * dz
    iota = jax.lax.broadcasted_iota(jnp.int32, (R, N), 1)
    kiota = jax.lax.broadcasted_iota(jnp.int32, (R, _KMAX), 1)

    def body(t, acc):
        d2 = d2_ref[...]
        v = jnp.min(d2, axis=1, keepdims=True)
        ix = jnp.min(jnp.where(d2 == v, iota, N), axis=1, keepdims=True)
        d2_ref[...] = jnp.where(iota == ix, jnp.inf, d2)
        return jnp.where(kiota == t, ix, acc)

    acc = jnp.zeros((R, _KMAX), jnp.int32)
    out_ref[0] = jax.lax.fori_loop(0, _KMAX, body, acc)


def _knn_top128(xyz):
    B, N, _ = xyz.shape
    xyzT = jnp.transpose(xyz, (0, 2, 1))
    return pl.pallas_call(
        _knn_kernel,
        grid=(B, N // _ROWS),
        in_specs=[
            pl.BlockSpec((1, _ROWS, 3), lambda b, i: (b, i, 0)),
            pl.BlockSpec((1, 3, N), lambda b, i: (b, 0, 0)),
        ],
        out_specs=pl.BlockSpec((1, _ROWS, _KMAX), lambda b, i: (b, i, 0)),
        out_shape=jax.ShapeDtypeStruct((B, N, _KMAX), jnp.int32),
        scratch_shapes=[pltpu.VMEM((_ROWS, N), jnp.float32)],
    )(xyz, xyzT)


def _mm_kernel(x_ref, w_ref, o_ref):
    o_ref[...] = jnp.dot(x_ref[...], w_ref[...],
                         preferred_element_type=jnp.float32)


def _mm(x, w):
    P, C = x.shape
    O = w.shape[1]
    BLK = 1024
    return pl.pallas_call(
        _mm_kernel,
        grid=(P // BLK,),
        in_specs=[
            pl.BlockSpec((BLK, C), lambda i: (i, 0)),
            pl.BlockSpec((C, O), lambda i: (0, 0)),
        ],
        out_specs=pl.BlockSpec((BLK, O), lambda i: (i, 0)),
        out_shape=jax.ShapeDtypeStruct((P, O), jnp.float32),
    )(x, w)


def _sc_gather(table, idx):
    R = idx.shape[0]
    per_w = R // _NW
    n_chunks = per_w // _CHUNK
    mesh = plsc.VectorSubcoreMesh(core_axis_name="c", subcore_axis_name="s")

    @functools.partial(
        pl.kernel,
        out_type=jax.ShapeDtypeStruct((R, 32), jnp.float32),
        mesh=mesh,
        compiler_params=pltpu.CompilerParams(use_tc_tiling_on_sc=False),
        scratch_types=[
            pltpu.VMEM((per_w,), jnp.int32),
            pltpu.VMEM((_CHUNK, 32), jnp.float32),
            pltpu.SemaphoreType.DMA,
        ],
    )
    def gather_k(table_hbm, idx_hbm, out_hbm, idx_v, buf, sem):
        wid = jax.lax.axis_index("s") * 2 + jax.lax.axis_index("c")
        base = wid * per_w
        pltpu.sync_copy(idx_hbm.at[pl.ds(base, per_w)], idx_v)

        def body(c, carry):
            pltpu.async_copy(
                table_hbm.at[idx_v.at[pl.ds(c * _CHUNK, _CHUNK)]], buf, sem
            ).wait()
            pltpu.sync_copy(buf, out_hbm.at[pl.ds(base + c * _CHUNK, _CHUNK)])
            return carry

        jax.lax.fori_loop(0, n_chunks, body, 0)

    return gather_k(table, idx)


def _reduce_kernel(g_ref, y_ref, m_ref, s_ref, q_ref):
    y = y_ref[...]
    h = g_ref[0] - y
    m, s, q = h, h, h * h
    for k in range(1, _K):
        h = g_ref[k] - y
        m = jnp.maximum(m, h)
        s = s + h
        q = q + h * h
    m_ref[...] = m
    s_ref[...] = s
    q_ref[...] = q


def _nbhd_reduce(g, y):
    P = y.shape[0]
    BLK = 1024
    sds = jax.ShapeDtypeStruct((P, 32), jnp.float32)
    return pl.pallas_call(
        _reduce_kernel,
        grid=(P // BLK,),
        in_specs=[
            pl.BlockSpec((_K, BLK, 32), lambda i: (0, i, 0)),
            pl.BlockSpec((BLK, 32), lambda i: (i, 0)),
        ],
        out_specs=[pl.BlockSpec((BLK, 32), lambda i: (i, 0))] * 3,
        out_shape=[sds, sds, sds],
    )(g, y)


def kernel(xyz, features, params):
    B, N, _ = xyz.shape
    P = B * N
    idx = _knn_top128(xyz)
    bbase = (jnp.arange(B, dtype=jnp.int32) * N)[:, None, None]

    def sel_flat(positions, table_off):
        p = jnp.array(positions, dtype=jnp.int32)
        s = idx[:, :, p] + bbase
        return s.reshape(P, _K).T + table_off

    feat_g = features
    feat_a = features
    graph_list = []
    ann_list = []
    cnt = jnp.float32(P * _K)
    for i, r in enumerate(_RATES):
        yg = _mm(feat_g.reshape(P, -1), params['dg_W%d' % i].T)
        ya = _mm(feat_a.reshape(P, -1), params['ad_W%d' % i].T)
        Y = jnp.concatenate([yg, ya], axis=0)
        IDX = jnp.concatenate(
            [sel_flat(_graph_positions(r), 0),
             sel_flat(_ann_positions(r), P)], axis=1)
        G = _sc_gather(Y, IDX.reshape(-1)).reshape(_K, 2 * P, 32)
        m, s, q = _nbhd_reduce(G, Y)
        outs = []
        for half, (gn, bn) in enumerate(
                [(params['dg_g%d' % i], params['dg_be%d' % i]),
                 (params['ad_g%d' % i], params['ad_be%d' % i])]):
            mh = m[half * P:(half + 1) * P]
            mean = jnp.sum(s[half * P:(half + 1) * P], axis=0) / cnt
            var = jnp.sum(q[half * P:(half + 1) * P], axis=0) / cnt - mean * mean
            f = jax.nn.relu((mh - mean) / jnp.sqrt(var + 1e-5) * gn + bn)
            outs.append(f.reshape(B, N, 32))
        graph_list.append(outs[0])
        ann_list.append(outs[1])
        feat_g = jnp.concatenate([feat_g, outs[0]], axis=-1)
        feat_a = jnp.concatenate([feat_a, outs[1]], axis=-1)

    fusion = jnp.concatenate(graph_list + ann_list, axis=-1)
    z = _mm(fusion.reshape(P, 2 * _OUTCH), params['fuse_W'].T)
    z = z.reshape(B, N, _OUTCH)
    mean = jnp.mean(z, axis=(0, 1))
    var = jnp.var(z, axis=(0, 1))
    h = (z - mean) / jnp.sqrt(var + 1e-5) * params['fuse_g'] + params['fuse_be']
    return jax.nn.relu(h)


# speedup vs baseline: 5.5304x; 1.2119x over previous
import functools
import math

import jax
import jax.numpy as jnp
from jax.experimental import pallas as pl
from jax.experimental.pallas import tpu as pltpu
from jax.experimental.pallas import tpu_sc as plsc

_RATES = [1, 2, 4, 8]
_OUTCH = 128
_K1 = 16
_STEP = 4
_KMAX = 128
_K = 16
_NW = 32
_CHUNK = 128


def _graph_positions(r):
    sn = (_K1 // _STEP) * (r - 1 + _STEP)
    n_iter = math.ceil(sn // (r - 1 + _STEP))
    pos = []
    for i in range(n_iter):
        lo = (i + 1) * (r - 1) + i * _STEP
        hi = sn if i == n_iter - 1 else (i + 1) * (r - 1 + _STEP)
        pos.extend(range(lo, hi))
    return pos


def _ann_positions(r):
    if r == 1:
        return list(range(16))
    return [0] + list(range((r - 1) * 16, r * 16 - 1))


_ROWS = 256
_LO = 63
_HI = 112
_SKIP = _HI - _LO


def _knn_kernel(xq_ref, xat_ref, out_ref, d2_ref):
    R = xq_ref.shape[1]
    N = xat_ref.shape[2]
    dx = xq_ref[0, :, 0:1] - xat_ref[0, 0:1, :]
    dy = xq_ref[0, :, 1:2] - xat_ref[0, 1:2, :]
    dz = xq_ref[0, :, 2:3] - xat_ref[0, 2:3, :]
    d2_ref[...] = dx * dx + dy * dy + dz * dz
    iota = jax.lax.broadcasted_iota(jnp.int32, (R, N), 1)
    kiota = jax.lax.broadcasted_iota(jnp.int32, (R, _KMAX), 1)

    def body(t, acc):
        d2 = d2_ref[...]
        v = jnp.min(d2, axis=1, keepdims=True)
        ix = jnp.min(jnp.where(d2 == v, iota, N), axis=1, keepdims=True)
        d2_ref[...] = jnp.where(iota == ix, jnp.inf, d2)
        return jnp.where(kiota == t, ix, acc)

    acc = jnp.zeros((R, _KMAX), jnp.int32)
    acc = jax.lax.fori_loop(0, _LO, body, acc)

    def bit_body(b, p):
        t2 = p | (jnp.int32(1) << (30 - b))
        bits = jax.lax.bitcast_convert_type(d2_ref[...], jnp.int32)
        c = jnp.sum((bits < t2).astype(jnp.int32), axis=1, keepdims=True)
        return jnp.where(c >= _SKIP, p, t2)

    V = jax.lax.fori_loop(0, 31, bit_body, jnp.zeros((R, 1), jnp.int32))
    bits = jax.lax.bitcast_convert_type(d2_ref[...], jnp.int32)
    removed = jnp.sum((bits < V).astype(jnp.int32), axis=1, keepdims=True)
    d2_ref[...] = jnp.where(bits < V, jnp.inf, d2_ref[...])

    def mop_cond(st):
        return jnp.any(st[0] < _SKIP)

    def mop(st):
        rem, _ = st
        d2 = d2_ref[...]
        b2 = jax.lax.bitcast_convert_type(d2, jnp.int32)
        ix = jnp.min(jnp.where((b2 == V) & (rem < _SKIP), iota, 1 << 30),
                     axis=1, keepdims=True)
        d2_ref[...] = jnp.where(iota == ix, jnp.inf, d2)
        return rem + (ix < (1 << 30)).astype(jnp.int32), 0

    removed, _ = jax.lax.while_loop(mop_cond, mop, (removed, 0))

    out_ref[0] = jax.lax.fori_loop(_HI, 127, body, acc)


def _knn_top128(xyz):
    B, N, _ = xyz.shape
    xyzT = jnp.transpose(xyz, (0, 2, 1))
    return pl.pallas_call(
        _knn_kernel,
        grid=(B, N // _ROWS),
        in_specs=[
            pl.BlockSpec((1, _ROWS, 3), lambda b, i: (b, i, 0)),
            pl.BlockSpec((1, 3, N), lambda b, i: (b, 0, 0)),
        ],
        out_specs=pl.BlockSpec((1, _ROWS, _KMAX), lambda b, i: (b, i, 0)),
        out_shape=jax.ShapeDtypeStruct((B, N, _KMAX), jnp.int32),
        scratch_shapes=[pltpu.VMEM((_ROWS, N), jnp.float32)],
    )(xyz, xyzT)


def _mm_kernel(x_ref, w_ref, o_ref):
    o_ref[...] = jnp.dot(x_ref[...], w_ref[...],
                         preferred_element_type=jnp.float32)


def _mm(x, w):
    P, C = x.shape
    O = w.shape[1]
    BLK = 1024
    return pl.pallas_call(
        _mm_kernel,
        grid=(P // BLK,),
        in_specs=[
            pl.BlockSpec((BLK, C), lambda i: (i, 0)),
            pl.BlockSpec((C, O), lambda i: (0, 0)),
        ],
        out_specs=pl.BlockSpec((BLK, O), lambda i: (i, 0)),
        out_shape=jax.ShapeDtypeStruct((P, O), jnp.float32),
    )(x, w)


def _mm2_kernel(x_ref, w_ref, o_ref):
    o_ref[0] = jnp.dot(x_ref[0], w_ref[0],
                       preferred_element_type=jnp.float32)


def _mm2(x, w):
    _, P, C = x.shape
    O = w.shape[2]
    BLK = 1024
    return pl.pallas_call(
        _mm2_kernel,
        grid=(2, P // BLK),
        in_specs=[
            pl.BlockSpec((1, BLK, C), lambda j, i: (j, i, 0)),
            pl.BlockSpec((1, C, O), lambda j, i: (j, 0, 0)),
        ],
        out_specs=pl.BlockSpec((1, BLK, O), lambda j, i: (j, i, 0)),
        out_shape=jax.ShapeDtypeStruct((2, P, O), jnp.float32),
    )(x, w)


def _sc_gather(table, idx):
    R = idx.shape[0]
    per_w = R // _NW
    n_chunks = per_w // _CHUNK
    mesh = plsc.VectorSubcoreMesh(core_axis_name="c", subcore_axis_name="s")

    NB = 4

    @functools.partial(
        pl.kernel,
        out_type=jax.ShapeDtypeStruct((R, 32), jnp.float32),
        mesh=mesh,
        compiler_params=pltpu.CompilerParams(use_tc_tiling_on_sc=False),
        scratch_types=[
            pltpu.VMEM((per_w,), jnp.int32),
        ] + [pltpu.VMEM((_CHUNK, 32), jnp.float32)] * NB
          + [pltpu.SemaphoreType.DMA] * NB,
    )
    def gather_k(table_hbm, idx_hbm, out_hbm, idx_v, *bufsem):
        bufs, sems = bufsem[:NB], bufsem[NB:]
        wid = jax.lax.axis_index("s") * 2 + jax.lax.axis_index("c")
        base = wid * per_w
        pltpu.sync_copy(idx_hbm.at[pl.ds(base, per_w)], idx_v)

        def body(i, carry):
            c = i * NB
            for j in range(NB):
                pltpu.async_copy(
                    table_hbm.at[idx_v.at[pl.ds((c + j) * _CHUNK, _CHUNK)]],
                    bufs[j], sems[j]).wait()
                pltpu.sync_copy(
                    bufs[j], out_hbm.at[pl.ds(base + (c + j) * _CHUNK, _CHUNK)])
            return carry

        jax.lax.fori_loop(0, n_chunks // NB, body, 0)

    return gather_k(table, idx)


def _reduce_kernel(g_ref, y_ref, m_ref, s_ref, q_ref):
    y = y_ref[...]
    h = g_ref[0] - y
    m, s, q = h, h, h * h
    for k in range(1, _K):
        h = g_ref[k] - y
        m = jnp.maximum(m, h)
        s = s + h
        q = q + h * h
    m_ref[...] = m
    s_ref[...] = s
    q_ref[...] = q


def _nbhd_reduce(g, y):
    P = y.shape[0]
    BLK = 1024
    sds = jax.ShapeDtypeStruct((P, 32), jnp.float32)
    return pl.pallas_call(
        _reduce_kernel,
        grid=(P // BLK,),
        in_specs=[
            pl.BlockSpec((_K, BLK, 32), lambda i: (0, i, 0)),
            pl.BlockSpec((BLK, 32), lambda i: (i, 0)),
        ],
        out_specs=[pl.BlockSpec((BLK, 32), lambda i: (i, 0))] * 3,
        out_shape=[sds, sds, sds],
    )(g, y)


def kernel(xyz, features, params):
    B, N, _ = xyz.shape
    P = B * N
    idx = _knn_top128(xyz)
    bbase = (jnp.arange(B, dtype=jnp.int32) * N)[:, None, None]

    def sel_flat(positions, table_off):
        p = jnp.array(positions, dtype=jnp.int32)
        s = idx[:, :, p] + bbase
        return s.reshape(P, _K).T + table_off

    feat_g = features
    feat_a = features
    graph_list = []
    ann_list = []
    cnt = jnp.float32(P * _K)
    for i, r in enumerate(_RATES):
        x2 = jnp.stack([feat_g.reshape(P, -1), feat_a.reshape(P, -1)])
        w2 = jnp.stack([params['dg_W%d' % i].T, params['ad_W%d' % i].T])
        Y = _mm2(x2, w2).reshape(2 * P, 32)
        IDX = jnp.concatenate(
            [sel_flat(_graph_positions(r), 0),
             sel_flat(_ann_positions(r), P)], axis=1)
        G = _sc_gather(Y, IDX.reshape(-1)).reshape(_K, 2 * P, 32)
        m, s, q = _nbhd_reduce(G, Y)
        outs = []
        for half, (gn, bn) in enumerate(
                [(params['dg_g%d' % i], params['dg_be%d' % i]),
                 (params['ad_g%d' % i], params['ad_be%d' % i])]):
            mh = m[half * P:(half + 1) * P]
            mean = jnp.sum(s[half * P:(half + 1) * P], axis=0) / cnt
            var = jnp.sum(q[half * P:(half + 1) * P], axis=0) / cnt - mean * mean
            f = jax.nn.relu((mh - mean) / jnp.sqrt(var + 1e-5) * gn + bn)
            outs.append(f.reshape(B, N, 32))
        graph_list.append(outs[0])
        ann_list.append(outs[1])
        feat_g = jnp.concatenate([feat_g, outs[0]], axis=-1)
        feat_a = jnp.concatenate([feat_a, outs[1]], axis=-1)

    fusion = jnp.concatenate(graph_list + ann_list, axis=-1)
    z = _mm(fusion.reshape(P, 2 * _OUTCH), params['fuse_W'].T)
    z = z.reshape(B, N, _OUTCH)
    mean = jnp.mean(z, axis=(0, 1))
    var = jnp.var(z, axis=(0, 1))
    h = (z - mean) / jnp.sqrt(var + 1e-5) * params['fuse_g'] + params['fuse_be']
    return jax.nn.relu(h)


# speedup vs baseline: 5.7640x; 1.0422x over previous
import functools
import math

import jax
import jax.numpy as jnp
from jax.experimental import pallas as pl
from jax.experimental.pallas import tpu as pltpu
from jax.experimental.pallas import tpu_sc as plsc

_RATES = [1, 2, 4, 8]
_OUTCH = 128
_K1 = 16
_STEP = 4
_KMAX = 128
_K = 16
_NW = 32
_CHUNK = 128


def _graph_positions(r):
    sn = (_K1 // _STEP) * (r - 1 + _STEP)
    n_iter = math.ceil(sn // (r - 1 + _STEP))
    pos = []
    for i in range(n_iter):
        lo = (i + 1) * (r - 1) + i * _STEP
        hi = sn if i == n_iter - 1 else (i + 1) * (r - 1 + _STEP)
        pos.extend(range(lo, hi))
    return pos


def _ann_positions(r):
    if r == 1:
        return list(range(16))
    return [0] + list(range((r - 1) * 16, r * 16 - 1))


_ROWS = 256
_LO = 63
_HI = 112
_SKIP = _HI - _LO


def _knn_kernel(xq_ref, xat_ref, out_ref, d2_ref):
    R = xq_ref.shape[1]
    N = xat_ref.shape[2]
    dx = xq_ref[0, :, 0:1] - xat_ref[0, 0:1, :]
    dy = xq_ref[0, :, 1:2] - xat_ref[0, 1:2, :]
    dz = xq_ref[0, :, 2:3] - xat_ref[0, 2:3, :]
    d2_ref[...] = dx * dx + dy * dy + dz * dz
    iota = jax.lax.broadcasted_iota(jnp.int32, (R, N), 1)
    kiota = jax.lax.broadcasted_iota(jnp.int32, (R, _KMAX), 1)

    def body(t, acc):
        d2 = d2_ref[...]
        v = jnp.min(d2, axis=1, keepdims=True)
        ix = jnp.min(jnp.where(d2 == v, iota, N), axis=1, keepdims=True)
        d2_ref[...] = jnp.where(iota == ix, jnp.inf, d2)
        return jnp.where(kiota == t, ix, acc)

    acc = jnp.zeros((R, _KMAX), jnp.int32)
    acc = jax.lax.fori_loop(0, _LO, body, acc)

    def bit_body(b, p):
        t2 = p | (jnp.int32(1) << (30 - b))
        bits = jax.lax.bitcast_convert_type(d2_ref[...], jnp.int32)
        c = jnp.sum((bits < t2).astype(jnp.int32), axis=1, keepdims=True)
        return jnp.where(c >= _SKIP, p, t2)

    V = jax.lax.fori_loop(0, 31, bit_body, jnp.zeros((R, 1), jnp.int32))
    bits = jax.lax.bitcast_convert_type(d2_ref[...], jnp.int32)
    removed = jnp.sum((bits < V).astype(jnp.int32), axis=1, keepdims=True)
    d2_ref[...] = jnp.where(bits < V, jnp.inf, d2_ref[...])

    def mop_cond(st):
        return jnp.any(st[0] < _SKIP)

    def mop(st):
        rem, _ = st
        d2 = d2_ref[...]
        b2 = jax.lax.bitcast_convert_type(d2, jnp.int32)
        ix = jnp.min(jnp.where((b2 == V) & (rem < _SKIP), iota, 1 << 30),
                     axis=1, keepdims=True)
        d2_ref[...] = jnp.where(iota == ix, jnp.inf, d2)
        return rem + (ix < (1 << 30)).astype(jnp.int32), 0

    removed, _ = jax.lax.while_loop(mop_cond, mop, (removed, 0))

    out_ref[0] = jax.lax.fori_loop(_HI, 127, body, acc)


def _knn_top128(xyz):
    B, N, _ = xyz.shape
    xyzT = jnp.transpose(xyz, (0, 2, 1))
    return pl.pallas_call(
        _knn_kernel,
        grid=(B, N // _ROWS),
        in_specs=[
            pl.BlockSpec((1, _ROWS, 3), lambda b, i: (b, i, 0)),
            pl.BlockSpec((1, 3, N), lambda b, i: (b, 0, 0)),
        ],
        out_specs=pl.BlockSpec((1, _ROWS, _KMAX), lambda b, i: (b, i, 0)),
        out_shape=jax.ShapeDtypeStruct((B, N, _KMAX), jnp.int32),
        scratch_shapes=[pltpu.VMEM((_ROWS, N), jnp.float32)],
    )(xyz, xyzT)


def _mm_kernel(x_ref, w_ref, o_ref):
    o_ref[...] = jnp.dot(x_ref[...], w_ref[...],
                         preferred_element_type=jnp.float32)


def _mm(x, w):
    P, C = x.shape
    O = w.shape[1]
    BLK = 1024
    return pl.pallas_call(
        _mm_kernel,
        grid=(P // BLK,),
        in_specs=[
            pl.BlockSpec((BLK, C), lambda i: (i, 0)),
            pl.BlockSpec((C, O), lambda i: (0, 0)),
        ],
        out_specs=pl.BlockSpec((BLK, O), lambda i: (i, 0)),
        out_shape=jax.ShapeDtypeStruct((P, O), jnp.float32),
    )(x, w)


def _mm2_kernel(x_ref, w_ref, o_ref):
    o_ref[0] = jnp.dot(x_ref[0], w_ref[0],
                       preferred_element_type=jnp.float32)


def _mm2(x, w):
    _, P, C = x.shape
    O = w.shape[2]
    BLK = 1024
    return pl.pallas_call(
        _mm2_kernel,
        grid=(2, P // BLK),
        in_specs=[
            pl.BlockSpec((1, BLK, C), lambda j, i: (j, i, 0)),
            pl.BlockSpec((1, C, O), lambda j, i: (j, 0, 0)),
        ],
        out_specs=pl.BlockSpec((1, BLK, O), lambda j, i: (j, i, 0)),
        out_shape=jax.ShapeDtypeStruct((2, P, O), jnp.float32),
    )(x, w)


def _sc_gather(table, idx):
    R = idx.shape[0]
    per_w = R // _NW
    n_chunks = per_w // _CHUNK
    mesh = plsc.VectorSubcoreMesh(core_axis_name="c", subcore_axis_name="s")

    NB = 4

    @functools.partial(
        pl.kernel,
        out_type=jax.ShapeDtypeStruct((R, 32), jnp.float32),
        mesh=mesh,
        compiler_params=pltpu.CompilerParams(use_tc_tiling_on_sc=False),
        scratch_types=[
            pltpu.VMEM((per_w,), jnp.int32),
        ] + [pltpu.VMEM((_CHUNK, 32), jnp.float32)] * NB
          + [pltpu.SemaphoreType.DMA],
    )
    def gather_k(table_hbm, idx_hbm, out_hbm, idx_v, *bufsem):
        bufs, sem = bufsem[:NB], bufsem[NB]
        wid = jax.lax.axis_index("s") * 2 + jax.lax.axis_index("c")
        base = wid * per_w
        pltpu.sync_copy(idx_hbm.at[pl.ds(base, per_w)], idx_v)

        def body(i, carry):
            c = i * NB
            cps = [
                pltpu.async_copy(
                    table_hbm.at[idx_v.at[pl.ds((c + j) * _CHUNK, _CHUNK)]],
                    bufs[j], sem)
                for j in range(NB)
            ]
            for j in range(NB):
                cps[j].wait()
            for j in range(NB):
                pltpu.sync_copy(
                    bufs[j], out_hbm.at[pl.ds(base + (c + j) * _CHUNK, _CHUNK)])
            return carry

        jax.lax.fori_loop(0, n_chunks // NB, body, 0)

    return gather_k(table, idx)


def _reduce_kernel(g_ref, y_ref, m_ref, s_ref, q_ref):
    y = y_ref[...]
    h = g_ref[0] - y
    m, s, q = h, h, h * h
    for k in range(1, _K):
        h = g_ref[k] - y
        m = jnp.maximum(m, h)
        s = s + h
        q = q + h * h
    m_ref[...] = m
    s_ref[...] = s
    q_ref[...] = q


def _nbhd_reduce(g, y):
    P = y.shape[0]
    BLK = 1024
    sds = jax.ShapeDtypeStruct((P, 32), jnp.float32)
    return pl.pallas_call(
        _reduce_kernel,
        grid=(P // BLK,),
        in_specs=[
            pl.BlockSpec((_K, BLK, 32), lambda i: (0, i, 0)),
            pl.BlockSpec((BLK, 32), lambda i: (i, 0)),
        ],
        out_specs=[pl.BlockSpec((BLK, 32), lambda i: (i, 0))] * 3,
        out_shape=[sds, sds, sds],
    )(g, y)


def kernel(xyz, features, params):
    B, N, _ = xyz.shape
    P = B * N
    idx = _knn_top128(xyz)
    bbase = (jnp.arange(B, dtype=jnp.int32) * N)[:, None, None]

    def sel_flat(positions, table_off):
        p = jnp.array(positions, dtype=jnp.int32)
        s = idx[:, :, p] + bbase
        return s.reshape(P, _K).T + table_off

    feat_g = features
    feat_a = features
    graph_list = []
    ann_list = []
    cnt = jnp.float32(P * _K)
    for i, r in enumerate(_RATES):
        x2 = jnp.stack([feat_g.reshape(P, -1), feat_a.reshape(P, -1)])
        w2 = jnp.stack([params['dg_W%d' % i].T, params['ad_W%d' % i].T])
        Y = _mm2(x2, w2).reshape(2 * P, 32)
        IDX = jnp.concatenate(
            [sel_flat(_graph_positions(r), 0),
             sel_flat(_ann_positions(r), P)], axis=1)
        G = _sc_gather(Y, IDX.reshape(-1)).reshape(_K, 2 * P, 32)
        m, s, q = _nbhd_reduce(G, Y)
        outs = []
        for half, (gn, bn) in enumerate(
                [(params['dg_g%d' % i], params['dg_be%d' % i]),
                 (params['ad_g%d' % i], params['ad_be%d' % i])]):
            mh = m[half * P:(half + 1) * P]
            mean = jnp.sum(s[half * P:(half + 1) * P], axis=0) / cnt
            var = jnp.sum(q[half * P:(half + 1) * P], axis=0) / cnt - mean * mean
            f = jax.nn.relu((mh - mean) / jnp.sqrt(var + 1e-5) * gn + bn)
            outs.append(f.reshape(B, N, 32))
        graph_list.append(outs[0])
        ann_list.append(outs[1])
        feat_g = jnp.concatenate([feat_g, outs[0]], axis=-1)
        feat_a = jnp.concatenate([feat_a, outs[1]], axis=-1)

    fusion = jnp.concatenate(graph_list + ann_list, axis=-1)
    z = _mm(fusion.reshape(P, 2 * _OUTCH), params['fuse_W'].T)
    z = z.reshape(B, N, _OUTCH)
    mean = jnp.mean(z, axis=(0, 1))
    var = jnp.var(z, axis=(0, 1))
    h = (z - mean) / jnp.sqrt(var + 1e-5) * params['fuse_g'] + params['fuse_be']
    return jax.nn.relu(h)


# speedup vs baseline: 6.1357x; 1.0645x over previous
import functools
import math

import jax
import jax.numpy as jnp
from jax.experimental import pallas as pl
from jax.experimental.pallas import tpu as pltpu
from jax.experimental.pallas import tpu_sc as plsc

_RATES = [1, 2, 4, 8]
_OUTCH = 128
_K1 = 16
_STEP = 4
_KMAX = 128
_K = 16
_NW = 32
_CHUNK = 128


def _graph_positions(r):
    sn = (_K1 // _STEP) * (r - 1 + _STEP)
    n_iter = math.ceil(sn // (r - 1 + _STEP))
    pos = []
    for i in range(n_iter):
        lo = (i + 1) * (r - 1) + i * _STEP
        hi = sn if i == n_iter - 1 else (i + 1) * (r - 1 + _STEP)
        pos.extend(range(lo, hi))
    return pos


def _ann_positions(r):
    if r == 1:
        return list(range(16))
    return [0] + list(range((r - 1) * 16, r * 16 - 1))


_ROWS = 512
_LO = 63
_HI = 112
_SKIP = _HI - _LO


def _knn_kernel(xq_ref, xat_ref, out_ref, d2_ref):
    R = xq_ref.shape[1]
    N = xat_ref.shape[2]
    dx = xq_ref[0, :, 0:1] - xat_ref[0, 0:1, :]
    dy = xq_ref[0, :, 1:2] - xat_ref[0, 1:2, :]
    dz = xq_ref[0, :, 2:3] - xat_ref[0, 2:3, :]
    d2_ref[...] = dx * dx + dy * dy + dz * dz
    iota = jax.lax.broadcasted_iota(jnp.int32, (R, N), 1)
    kiota = jax.lax.broadcasted_iota(jnp.int32, (R, _KMAX), 1)

    def body(t, acc):
        d2 = d2_ref[...]
        v = jnp.min(d2, axis=1, keepdims=True)
        ix = jnp.min(jnp.where(d2 == v, iota, N), axis=1, keepdims=True)
        d2_ref[...] = jnp.where(iota == ix, jnp.inf, d2)
        return jnp.where(kiota == t, ix, acc)

    acc = jnp.zeros((R, _KMAX), jnp.int32)
    acc = jax.lax.fori_loop(0, _LO, body, acc)

    def bit_body(b, p):
        t2 = p | (jnp.int32(1) << (30 - b))
        bits = jax.lax.bitcast_convert_type(d2_ref[...], jnp.int32)
        c = jnp.sum((bits < t2).astype(jnp.int32), axis=1, keepdims=True)
        return jnp.where(c >= _SKIP, p, t2)

    V = jax.lax.fori_loop(0, 31, bit_body, jnp.zeros((R, 1), jnp.int32))
    bits = jax.lax.bitcast_convert_type(d2_ref[...], jnp.int32)
    removed = jnp.sum((bits < V).astype(jnp.int32), axis=1, keepdims=True)
    d2_ref[...] = jnp.where(bits < V, jnp.inf, d2_ref[...])

    def mop_cond(st):
        return jnp.any(st[0] < _SKIP)

    def mop(st):
        rem, _ = st
        d2 = d2_ref[...]
        b2 = jax.lax.bitcast_convert_type(d2, jnp.int32)
        ix = jnp.min(jnp.where((b2 == V) & (rem < _SKIP), iota, 1 << 30),
                     axis=1, keepdims=True)
        d2_ref[...] = jnp.where(iota == ix, jnp.inf, d2)
        return rem + (ix < (1 << 30)).astype(jnp.int32), 0

    removed, _ = jax.lax.while_loop(mop_cond, mop, (removed, 0))

    out_ref[0] = jax.lax.fori_loop(_HI, 127, body, acc)


def _knn_top128(xyz):
    B, N, _ = xyz.shape
    xyzT = jnp.transpose(xyz, (0, 2, 1))
    return pl.pallas_call(
        _knn_kernel,
        grid=(B, N // _ROWS),
        in_specs=[
            pl.BlockSpec((1, _ROWS, 3), lambda b, i: (b, i, 0)),
            pl.BlockSpec((1, 3, N), lambda b, i: (b, 0, 0)),
        ],
        out_specs=pl.BlockSpec((1, _ROWS, _KMAX), lambda b, i: (b, i, 0)),
        out_shape=jax.ShapeDtypeStruct((B, N, _KMAX), jnp.int32),
        scratch_shapes=[pltpu.VMEM((_ROWS, N), jnp.float32)],
    )(xyz, xyzT)


def _mm_kernel(x_ref, w_ref, o_ref):
    o_ref[...] = jnp.dot(x_ref[...], w_ref[...],
                         preferred_element_type=jnp.float32)


def _mm(x, w):
    P, C = x.shape
    O = w.shape[1]
    BLK = 1024
    return pl.pallas_call(
        _mm_kernel,
        grid=(P // BLK,),
        in_specs=[
            pl.BlockSpec((BLK, C), lambda i: (i, 0)),
            pl.BlockSpec((C, O), lambda i: (0, 0)),
        ],
        out_specs=pl.BlockSpec((BLK, O), lambda i: (i, 0)),
        out_shape=jax.ShapeDtypeStruct((P, O), jnp.float32),
    )(x, w)


def _mm2_kernel(x_ref, w_ref, o_ref):
    o_ref[0] = jnp.dot(x_ref[0], w_ref[0],
                       preferred_element_type=jnp.float32)


def _mm2(x, w):
    _, P, C = x.shape
    O = w.shape[2]
    BLK = 1024
    return pl.pallas_call(
        _mm2_kernel,
        grid=(2, P // BLK),
        in_specs=[
            pl.BlockSpec((1, BLK, C), lambda j, i: (j, i, 0)),
            pl.BlockSpec((1, C, O), lambda j, i: (j, 0, 0)),
        ],
        out_specs=pl.BlockSpec((1, BLK, O), lambda j, i: (j, i, 0)),
        out_shape=jax.ShapeDtypeStruct((2, P, O), jnp.float32),
    )(x, w)


def _sc_gather(table, idx):
    R = idx.shape[0]
    per_w = R // _NW
    n_chunks = per_w // _CHUNK
    mesh = plsc.VectorSubcoreMesh(core_axis_name="c", subcore_axis_name="s")

    NB = 8

    @functools.partial(
        pl.kernel,
        out_type=jax.ShapeDtypeStruct((R, 32), jnp.float32),
        mesh=mesh,
        compiler_params=pltpu.CompilerParams(use_tc_tiling_on_sc=False),
        scratch_types=[
            pltpu.VMEM((per_w,), jnp.int32),
        ] + [pltpu.VMEM((_CHUNK, 32), jnp.float32)] * NB
          + [pltpu.SemaphoreType.DMA],
    )
    def gather_k(table_hbm, idx_hbm, out_hbm, idx_v, *bufsem):
        bufs, sem = bufsem[:NB], bufsem[NB]
        wid = jax.lax.axis_index("s") * 2 + jax.lax.axis_index("c")
        base = wid * per_w
        pltpu.sync_copy(idx_hbm.at[pl.ds(base, per_w)], idx_v)

        def body(i, carry):
            c = i * NB
            cps = [
                pltpu.async_copy(
                    table_hbm.at[idx_v.at[pl.ds((c + j) * _CHUNK, _CHUNK)]],
                    bufs[j], sem)
                for j in range(NB)
            ]
            for j in range(NB):
                cps[j].wait()
            for j in range(NB):
                pltpu.sync_copy(
                    bufs[j], out_hbm.at[pl.ds(base + (c + j) * _CHUNK, _CHUNK)])
            return carry

        jax.lax.fori_loop(0, n_chunks // NB, body, 0)

    return gather_k(table, idx)


def _reduce_kernel(g_ref, y_ref, m_ref, s_ref, q_ref):
    y = y_ref[...]
    h = g_ref[0] - y
    m, s, q = h, h, h * h
    for k in range(1, _K):
        h = g_ref[k] - y
        m = jnp.maximum(m, h)
        s = s + h
        q = q + h * h
    m_ref[...] = m
    s_ref[...] = s
    q_ref[...] = q


def _nbhd_reduce(g, y):
    P = y.shape[0]
    BLK = 1024
    sds = jax.ShapeDtypeStruct((P, 32), jnp.float32)
    return pl.pallas_call(
        _reduce_kernel,
        grid=(P // BLK,),
        in_specs=[
            pl.BlockSpec((_K, BLK, 32), lambda i: (0, i, 0)),
            pl.BlockSpec((BLK, 32), lambda i: (i, 0)),
        ],
        out_specs=[pl.BlockSpec((BLK, 32), lambda i: (i, 0))] * 3,
        out_shape=[sds, sds, sds],
    )(g, y)


def kernel(xyz, features, params):
    B, N, _ = xyz.shape
    P = B * N
    idx = _knn_top128(xyz)
    bbase = (jnp.arange(B, dtype=jnp.int32) * N)[:, None, None]

    def sel_flat(positions, table_off):
        p = jnp.array(positions, dtype=jnp.int32)
        s = idx[:, :, p] + bbase
        return s.reshape(P, _K).T + table_off

    feat_g = features
    feat_a = features
    graph_list = []
    ann_list = []
    cnt = jnp.float32(P * _K)
    for i, r in enumerate(_RATES):
        x2 = jnp.stack([feat_g.reshape(P, -1), feat_a.reshape(P, -1)])
        w2 = jnp.stack([params['dg_W%d' % i].T, params['ad_W%d' % i].T])
        Y = _mm2(x2, w2).reshape(2 * P, 32)
        IDX = jnp.concatenate(
            [sel_flat(_graph_positions(r), 0),
             sel_flat(_ann_positions(r), P)], axis=1)
        G = _sc_gather(Y, IDX.reshape(-1)).reshape(_K, 2 * P, 32)
        m, s, q = _nbhd_reduce(G, Y)
        outs = []
        for half, (gn, bn) in enumerate(
                [(params['dg_g%d' % i], params['dg_be%d' % i]),
                 (params['ad_g%d' % i], params['ad_be%d' % i])]):
            mh = m[half * P:(half + 1) * P]
            mean = jnp.sum(s[half * P:(half + 1) * P], axis=0) / cnt
            var = jnp.sum(q[half * P:(half + 1) * P], axis=0) / cnt - mean * mean
            f = jax.nn.relu((mh - mean) / jnp.sqrt(var + 1e-5) * gn + bn)
            outs.append(f.reshape(B, N, 32))
        graph_list.append(outs[0])
        ann_list.append(outs[1])
        feat_g = jnp.concatenate([feat_g, outs[0]], axis=-1)
        feat_a = jnp.concatenate([feat_a, outs[1]], axis=-1)

    fusion = jnp.concatenate(graph_list + ann_list, axis=-1)
    z = _mm(fusion.reshape(P, 2 * _OUTCH), params['fuse_W'].T)
    z = z.reshape(B, N, _OUTCH)
    mean = jnp.mean(z, axis=(0, 1))
    var = jnp.var(z, axis=(0, 1))
    h = (z - mean) / jnp.sqrt(var + 1e-5) * params['fuse_g'] + params['fuse_be']
    return jax.nn.relu(h)


# speedup vs baseline: 6.2772x; 1.0231x over previous
import functools
import math

import jax
import jax.numpy as jnp
from jax.experimental import pallas as pl
from jax.experimental.pallas import tpu as pltpu
from jax.experimental.pallas import tpu_sc as plsc

_RATES = [1, 2, 4, 8]
_OUTCH = 128
_K1 = 16
_STEP = 4
_KMAX = 128
_K = 16
_NW = 32
_CHUNK = 128


def _graph_positions(r):
    sn = (_K1 // _STEP) * (r - 1 + _STEP)
    n_iter = math.ceil(sn // (r - 1 + _STEP))
    pos = []
    for i in range(n_iter):
        lo = (i + 1) * (r - 1) + i * _STEP
        hi = sn if i == n_iter - 1 else (i + 1) * (r - 1 + _STEP)
        pos.extend(range(lo, hi))
    return pos


def _ann_positions(r):
    if r == 1:
        return list(range(16))
    return [0] + list(range((r - 1) * 16, r * 16 - 1))


_ROWS = 1024
_LO = 63
_HI = 112
_SKIP = _HI - _LO


def _knn_kernel(xq_ref, xat_ref, out_ref, d2_ref):
    R = xq_ref.shape[1]
    N = xat_ref.shape[2]
    dx = xq_ref[0, :, 0:1] - xat_ref[0, 0:1, :]
    dy = xq_ref[0, :, 1:2] - xat_ref[0, 1:2, :]
    dz = xq_ref[0, :, 2:3] - xat_ref[0, 2:3, :]
    d2_ref[...] = dx * dx + dy * dy + dz * dz
    iota = jax.lax.broadcasted_iota(jnp.int32, (R, N), 1)
    kiota = jax.lax.broadcasted_iota(jnp.int32, (R, _KMAX), 1)

    def body(t, acc):
        d2 = d2_ref[...]
        v = jnp.min(d2, axis=1, keepdims=True)
        ix = jnp.min(jnp.where(d2 == v, iota, N), axis=1, keepdims=True)
        d2_ref[...] = jnp.where(iota == ix, jnp.inf, d2)
        return jnp.where(kiota == t, ix, acc)

    acc = jnp.zeros((R, _KMAX), jnp.int32)
    acc = jax.lax.fori_loop(0, _LO, body, acc)

    def bit_body(b, p):
        t2 = p | (jnp.int32(1) << (30 - b))
        bits = jax.lax.bitcast_convert_type(d2_ref[...], jnp.int32)
        c = jnp.sum((bits < t2).astype(jnp.int32), axis=1, keepdims=True)
        return jnp.where(c >= _SKIP, p, t2)

    V = jax.lax.fori_loop(0, 31, bit_body, jnp.zeros((R, 1), jnp.int32))
    bits = jax.lax.bitcast_convert_type(d2_ref[...], jnp.int32)
    removed = jnp.sum((bits < V).astype(jnp.int32), axis=1, keepdims=True)
    d2_ref[...] = jnp.where(bits < V, jnp.inf, d2_ref[...])

    def mop_cond(st):
        return jnp.any(st[0] < _SKIP)

    def mop(st):
        rem, _ = st
        d2 = d2_ref[...]
        b2 = jax.lax.bitcast_convert_type(d2, jnp.int32)
        ix = jnp.min(jnp.where((b2 == V) & (rem < _SKIP), iota, 1 << 30),
                     axis=1, keepdims=True)
        d2_ref[...] = jnp.where(iota == ix, jnp.inf, d2)
        return rem + (ix < (1 << 30)).astype(jnp.int32), 0

    removed, _ = jax.lax.while_loop(mop_cond, mop, (removed, 0))

    out_ref[0] = jax.lax.fori_loop(_HI, 127, body, acc)


def _knn_top128(xyz):
    B, N, _ = xyz.shape
    xyzT = jnp.transpose(xyz, (0, 2, 1))
    return pl.pallas_call(
        _knn_kernel,
        grid=(B, N // _ROWS),
        in_specs=[
            pl.BlockSpec((1, _ROWS, 3), lambda b, i: (b, i, 0)),
            pl.BlockSpec((1, 3, N), lambda b, i: (b, 0, 0)),
        ],
        out_specs=pl.BlockSpec((1, _ROWS, _KMAX), lambda b, i: (b, i, 0)),
        out_shape=jax.ShapeDtypeStruct((B, N, _KMAX), jnp.int32),
        scratch_shapes=[pltpu.VMEM((_ROWS, N), jnp.float32)],
    )(xyz, xyzT)


def _mm_kernel(x_ref, w_ref, o_ref):
    o_ref[...] = jnp.dot(x_ref[...], w_ref[...],
                         preferred_element_type=jnp.float32)


def _mm(x, w):
    P, C = x.shape
    O = w.shape[1]
    BLK = 1024
    return pl.pallas_call(
        _mm_kernel,
        grid=(P // BLK,),
        in_specs=[
            pl.BlockSpec((BLK, C), lambda i: (i, 0)),
            pl.BlockSpec((C, O), lambda i: (0, 0)),
        ],
        out_specs=pl.BlockSpec((BLK, O), lambda i: (i, 0)),
        out_shape=jax.ShapeDtypeStruct((P, O), jnp.float32),
    )(x, w)


def _mm2_kernel(x_ref, w_ref, o_ref):
    o_ref[0] = jnp.dot(x_ref[0], w_ref[0],
                       preferred_element_type=jnp.float32)


def _mm2(x, w):
    _, P, C = x.shape
    O = w.shape[2]
    BLK = 1024
    return pl.pallas_call(
        _mm2_kernel,
        grid=(2, P // BLK),
        in_specs=[
            pl.BlockSpec((1, BLK, C), lambda j, i: (j, i, 0)),
            pl.BlockSpec((1, C, O), lambda j, i: (j, 0, 0)),
        ],
        out_specs=pl.BlockSpec((1, BLK, O), lambda j, i: (j, i, 0)),
        out_shape=jax.ShapeDtypeStruct((2, P, O), jnp.float32),
    )(x, w)


def _sc_gather(table, idx):
    R = idx.shape[0]
    per_w = R // _NW
    n_chunks = per_w // _CHUNK
    mesh = plsc.VectorSubcoreMesh(core_axis_name="c", subcore_axis_name="s")

    NB = 8

    @functools.partial(
        pl.kernel,
        out_type=jax.ShapeDtypeStruct((R, 32), jnp.float32),
        mesh=mesh,
        compiler_params=pltpu.CompilerParams(use_tc_tiling_on_sc=False),
        scratch_types=[
            pltpu.VMEM((per_w,), jnp.int32),
        ] + [pltpu.VMEM((_CHUNK, 32), jnp.float32)] * NB
          + [pltpu.SemaphoreType.DMA],
    )
    def gather_k(table_hbm, idx_hbm, out_hbm, idx_v, *bufsem):
        bufs, sem = bufsem[:NB], bufsem[NB]
        wid = jax.lax.axis_index("s") * 2 + jax.lax.axis_index("c")
        base = wid * per_w
        pltpu.sync_copy(idx_hbm.at[pl.ds(base, per_w)], idx_v)

        def body(i, carry):
            c = i * NB
            cps = [
                pltpu.async_copy(
                    table_hbm.at[idx_v.at[pl.ds((c + j) * _CHUNK, _CHUNK)]],
                    bufs[j], sem)
                for j in range(NB)
            ]
            for j in range(NB):
                cps[j].wait()
            for j in range(NB):
                pltpu.sync_copy(
                    bufs[j], out_hbm.at[pl.ds(base + (c + j) * _CHUNK, _CHUNK)])
            return carry

        jax.lax.fori_loop(0, n_chunks // NB, body, 0)

    return gather_k(table, idx)


def _reduce_kernel(g_ref, y_ref, m_ref, s_ref, q_ref):
    y = y_ref[...]
    h = g_ref[0] - y
    m, s, q = h, h, h * h
    for k in range(1, _K):
        h = g_ref[k] - y
        m = jnp.maximum(m, h)
        s = s + h
        q = q + h * h
    m_ref[...] = m
    s_ref[...] = s
    q_ref[...] = q


def _nbhd_reduce(g, y):
    P = y.shape[0]
    BLK = 1024
    sds = jax.ShapeDtypeStruct((P, 32), jnp.float32)
    return pl.pallas_call(
        _reduce_kernel,
        grid=(P // BLK,),
        in_specs=[
            pl.BlockSpec((_K, BLK, 32), lambda i: (0, i, 0)),
            pl.BlockSpec((BLK, 32), lambda i: (i, 0)),
        ],
        out_specs=[pl.BlockSpec((BLK, 32), lambda i: (i, 0))] * 3,
        out_shape=[sds, sds, sds],
    )(g, y)


def kernel(xyz, features, params):
    B, N, _ = xyz.shape
    P = B * N
    idx = _knn_top128(xyz)
    bbase = (jnp.arange(B, dtype=jnp.int32) * N)[:, None, None]

    def sel_flat(positions, table_off):
        p = jnp.array(positions, dtype=jnp.int32)
        s = idx[:, :, p] + bbase
        return s.reshape(P, _K).T + table_off

    feat_g = features
    feat_a = features
    graph_list = []
    ann_list = []
    cnt = jnp.float32(P * _K)
    for i, r in enumerate(_RATES):
        x2 = jnp.stack([feat_g.reshape(P, -1), feat_a.reshape(P, -1)])
        w2 = jnp.stack([params['dg_W%d' % i].T, params['ad_W%d' % i].T])
        Y = _mm2(x2, w2).reshape(2 * P, 32)
        IDX = jnp.concatenate(
            [sel_flat(_graph_positions(r), 0),
             sel_flat(_ann_positions(r), P)], axis=1)
        G = _sc_gather(Y, IDX.reshape(-1)).reshape(_K, 2 * P, 32)
        m, s, q = _nbhd_reduce(G, Y)
        outs = []
        for half, (gn, bn) in enumerate(
                [(params['dg_g%d' % i], params['dg_be%d' % i]),
                 (params['ad_g%d' % i], params['ad_be%d' % i])]):
            mh = m[half * P:(half + 1) * P]
            mean = jnp.sum(s[half * P:(half + 1) * P], axis=0) / cnt
            var = jnp.sum(q[half * P:(half + 1) * P], axis=0) / cnt - mean * mean
            f = jax.nn.relu((mh - mean) / jnp.sqrt(var + 1e-5) * gn + bn)
            outs.append(f.reshape(B, N, 32))
        graph_list.append(outs[0])
        ann_list.append(outs[1])
        feat_g = jnp.concatenate([feat_g, outs[0]], axis=-1)
        feat_a = jnp.concatenate([feat_a, outs[1]], axis=-1)

    fusion = jnp.concatenate(graph_list + ann_list, axis=-1)
    z = _mm(fusion.reshape(P, 2 * _OUTCH), params['fuse_W'].T)
    z = z.reshape(B, N, _OUTCH)
    mean = jnp.mean(z, axis=(0, 1))
    var = jnp.var(z, axis=(0, 1))
    h = (z - mean) / jnp.sqrt(var + 1e-5) * params['fuse_g'] + params['fuse_be']
    return jax.nn.relu(h)
